# Initial kernel scaffold; baseline (speedup 1.0000x reference)
import functools

import jax
import jax.numpy as jnp
from jax import lax
from jax.experimental import pallas as pl
from jax.experimental.pallas import tpu as pltpu
from jax.experimental.pallas import tpu_sc as plsc

NC, NS, L = 2, 16, 16
NW = NC * NS
K = 80
ZB = 125
BE = 2000


def _sc_mesh():
    return plsc.VectorSubcoreMesh(core_axis_name="c", subcore_axis_name="s")


def _sc_edge_features(px, py, rg, src, dst):
    n = px.shape[0]
    e = src.shape[0]
    per_w = e // NW
    n_chunks = per_w // K
    rows_n = n // NS

    @functools.partial(
        pl.kernel,
        out_type=(
            jax.ShapeDtypeStruct((e,), jnp.float32),
            jax.ShapeDtypeStruct((e,), jnp.float32),
            jax.ShapeDtypeStruct((e,), jnp.float32),
            jax.ShapeDtypeStruct((NC, n, 16), jnp.float32),
        ),
        mesh=_sc_mesh(),
        scratch_types=[
            pltpu.VMEM((n,), jnp.float32),
            pltpu.VMEM((n,), jnp.float32),
            pltpu.VMEM((n,), jnp.float32),
            pltpu.VMEM((K,), jnp.int32),
            pltpu.VMEM((K,), jnp.int32),
            pltpu.VMEM((K,), jnp.float32),
            pltpu.VMEM((K,), jnp.float32),
            pltpu.VMEM((K,), jnp.float32),
            pltpu.VMEM((K, 16), jnp.float32),
            pltpu.VMEM((ZB, 16), jnp.float32),
            pltpu.VMEM_SHARED((n, 16), jnp.float32),
            pltpu.SemaphoreType.DMA,
        ],
    )
    def kern(px_h, py_h, rg_h, src_h, dst_h, ex_h, ey_h, er_h, degp_h,
             px_v, py_v, rg_v, idx_s, idx_d, ex_v, ey_v, er_v, ones_v,
             zbuf, deg_sh, sem):
        cid = lax.axis_index("c")
        sid = lax.axis_index("s")
        wid = sid * NC + cid
        pltpu.sync_copy(px_h, px_v)
        pltpu.sync_copy(py_h, py_v)
        pltpu.sync_copy(rg_h, rg_v)
        zeros16 = jnp.zeros((L,), jnp.float32)
        ones16 = jnp.ones((L,), jnp.float32)

        @pl.loop(0, K)
        def _(i):
            ones_v[i, :] = ones16

        @pl.loop(0, ZB)
        def _(i):
            zbuf[i, :] = zeros16

        row0 = sid * rows_n

        @pl.loop(0, rows_n // ZB)
        def _(i):
            pltpu.sync_copy(zbuf, deg_sh.at[pl.ds(row0 + i * ZB, ZB), :])

        plsc.subcore_barrier()
        base_w = wid * per_w

        @pl.loop(0, n_chunks)
        def _(i):
            base = pl.multiple_of(base_w + i * K, 16)
            pltpu.sync_copy(src_h.at[pl.ds(base, K)], idx_s)
            pltpu.sync_copy(dst_h.at[pl.ds(base, K)], idx_d)
            for j in range(K // L):
                s = idx_s[pl.ds(j * L, L)]
                d = idx_d[pl.ds(j * L, L)]
                pxs = plsc.load_gather(px_v, [s])
                pxd = plsc.load_gather(px_v, [d])
                pys = plsc.load_gather(py_v, [s])
                pyd = plsc.load_gather(py_v, [d])
                rgs = plsc.load_gather(rg_v, [s])
                ex_v[pl.ds(j * L, L)] = pxs - pxd
                ey_v[pl.ds(j * L, L)] = pys - pyd
                er_v[pl.ds(j * L, L)] = rgs
            pltpu.sync_copy(ex_v, ex_h.at[pl.ds(base, K)])
            pltpu.sync_copy(ey_v, ey_h.at[pl.ds(base, K)])
            pltpu.sync_copy(er_v, er_h.at[pl.ds(base, K)])
            pltpu.sync_copy(ones_v, deg_sh.at[idx_d], add=True)

        plsc.subcore_barrier()
        pltpu.sync_copy(deg_sh.at[pl.ds(row0, rows_n), :],
                        degp_h.at[cid, pl.ds(row0, rows_n), :])

    return kern(px, py, rg, src, dst)


def _sc_msg_agg(h, gate, src, dst):
    n, hd = h.shape
    e = src.shape[0]
    per_w = e // NW
    n_chunks = per_w // K
    rows_n = n // NS
    hl = hd // L

    @functools.partial(
        pl.kernel,
        out_type=jax.ShapeDtypeStruct((NC, n, hd), jnp.float32),
        mesh=_sc_mesh(),
        scratch_types=[
            pltpu.VMEM((K,), jnp.int32),
            pltpu.VMEM((K,), jnp.int32),
            pltpu.VMEM((K, hd), jnp.float32),
            pltpu.VMEM((K, hd), jnp.float32),
            pltpu.VMEM((ZB, hd), jnp.float32),
            pltpu.VMEM_SHARED((n, hd), jnp.float32),
            pltpu.SemaphoreType.DMA,
        ],
    )
    def kern(h_h, gate_h, src_h, dst_h, out_h,
             idx_s, idx_d, rows_v, gate_v, zbuf, agg_sh, sem):
        cid = lax.axis_index("c")
        sid = lax.axis_index("s")
        wid = sid * NC + cid
        zeros16 = jnp.zeros((L,), jnp.float32)

        @pl.loop(0, ZB)
        def _(i):
            for j in range(hl):
                zbuf[i, pl.ds(j * L, L)] = zeros16

        row0 = sid * rows_n

        @pl.loop(0, rows_n // ZB)
        def _(i):
            pltpu.sync_copy(zbuf, agg_sh.at[pl.ds(row0 + i * ZB, ZB), :])

        plsc.subcore_barrier()
        base_w = wid * per_w

        @pl.loop(0, n_chunks)
        def _(i):
            base = pl.multiple_of(base_w + i * K, 16)
            pltpu.sync_copy(src_h.at[pl.ds(base, K)], idx_s)
            pltpu.sync_copy(dst_h.at[pl.ds(base, K)], idx_d)
            g = pltpu.async_copy(h_h.at[idx_s], rows_v, sem)
            pltpu.sync_copy(gate_h.at[pl.ds(base, K), :], gate_v)
            g.wait()

            @pl.loop(0, K)
            def _(r):
                for j in range(hl):
                    sl = pl.ds(j * L, L)
                    rows_v[r, sl] = rows_v[r, sl] * gate_v[r, sl]

            pltpu.sync_copy(rows_v, agg_sh.at[idx_d], add=True)

        plsc.subcore_barrier()
        pltpu.sync_copy(agg_sh.at[pl.ds(row0, rows_n), :],
                        out_h.at[cid, pl.ds(row0, rows_n), :])

    return kern(h, gate, src, dst)


def _sc_pair_gather_add(u, v, src, dst):
    n, hd = u.shape
    e = src.shape[0]
    per_w = e // NW
    n_chunks = per_w // K
    hl = hd // L

    @functools.partial(
        pl.kernel,
        out_type=jax.ShapeDtypeStruct((e, hd), jnp.float32),
        mesh=_sc_mesh(),
        scratch_types=[
            pltpu.VMEM((K,), jnp.int32),
            pltpu.VMEM((K,), jnp.int32),
            pltpu.VMEM((K, hd), jnp.float32),
            pltpu.VMEM((K, hd), jnp.float32),
            pltpu.SemaphoreType.DMA,
            pltpu.SemaphoreType.DMA,
        ],
    )
    def kern(u_h, v_h, src_h, dst_h, out_h,
             idx_s, idx_d, rows_u, rows_v, sem_u, sem_v):
        cid = lax.axis_index("c")
        sid = lax.axis_index("s")
        wid = sid * NC + cid
        base_w = wid * per_w

        @pl.loop(0, n_chunks)
        def _(i):
            base = pl.multiple_of(base_w + i * K, 16)
            pltpu.sync_copy(src_h.at[pl.ds(base, K)], idx_s)
            pltpu.sync_copy(dst_h.at[pl.ds(base, K)], idx_d)
            gu = pltpu.async_copy(u_h.at[idx_s], rows_u, sem_u)
            gv = pltpu.async_copy(v_h.at[idx_d], rows_v, sem_v)
            gu.wait()
            gv.wait()

            @pl.loop(0, K)
            def _(r):
                for j in range(hl):
                    sl = pl.ds(j * L, L)
                    rows_u[r, sl] = rows_u[r, sl] + rows_v[r, sl]

            pltpu.sync_copy(rows_u, out_h.at[pl.ds(base, K), :])

    return kern(u, v, src, dst)


def _tc_linear(x, w, b, relu=False):
    n = x.shape[0]
    m = w.shape[1]

    def body(x_ref, w_ref, b_ref, o_ref):
        y = jnp.dot(x_ref[...], w_ref[...],
                    preferred_element_type=jnp.float32) + b_ref[...]
        if relu:
            y = jnp.maximum(y, 0.0)
        o_ref[...] = y

    return pl.pallas_call(
        body, out_shape=jax.ShapeDtypeStruct((n, m), jnp.float32),
    )(x, w, b.reshape(1, -1))


def _tc_gates(ex, ey, er, w11, b11, w12, b12, w21, b21, w22, b22):
    e = ex.shape[0]
    hd = w12.shape[1]
    grid = e // BE
    exc = ex.reshape(e, 1)
    eyc = ey.reshape(e, 1)
    erc = er.reshape(e, 1)

    def body(ex_ref, ey_ref, er_ref, w11_r, b11_r, w12_r, b12_r,
             w21_r, b21_r, w22_r, b22_r, g1_ref, g2_ref):
        exb = ex_ref[...]
        eyb = ey_ref[...]
        erb = er_ref[...]
        pre1 = (exb * w11_r[0:1, :] + eyb * w11_r[1:2, :]
                + erb * w11_r[2:3, :] + b11_r[...])
        g1_ref[...] = jax.nn.sigmoid(
            jnp.dot(jnp.maximum(pre1, 0.0), w12_r[...],
                    preferred_element_type=jnp.float32) + b12_r[...])
        pre2 = (exb * w21_r[0:1, :] + eyb * w21_r[1:2, :]
                + erb * w21_r[2:3, :] + b21_r[...])
        g2_ref[...] = jax.nn.sigmoid(
            jnp.dot(jnp.maximum(pre2, 0.0), w22_r[...],
                    preferred_element_type=jnp.float32) + b22_r[...])

    col = pl.BlockSpec((BE, 1), lambda i: (i, 0))
    wfull = lambda a: pl.BlockSpec(a.shape, lambda i: (0,) * a.ndim)
    outspec = pl.BlockSpec((BE, hd), lambda i: (i, 0))
    b11c, b12c = b11.reshape(1, -1), b12.reshape(1, -1)
    b21c, b22c = b21.reshape(1, -1), b22.reshape(1, -1)
    return pl.pallas_call(
        body,
        grid=(grid,),
        in_specs=[col, col, col,
                  wfull(w11), wfull(b11c), wfull(w12), wfull(b12c),
                  wfull(w21), wfull(b21c), wfull(w22), wfull(b22c)],
        out_specs=[outspec, outspec],
        out_shape=[jax.ShapeDtypeStruct((e, hd), jnp.float32),
                   jax.ShapeDtypeStruct((e, hd), jnp.float32)],
    )(exc, eyc, erc, w11, b11c, w12, b12c, w21, b21c, w22, b22c)


def _tc_z(parts, degp):
    n, hd = parts.shape[1], parts.shape[2]

    def body(p_ref, d_ref, o_ref):
        deg = d_ref[0, :, 0:1] + d_ref[1, :, 0:1]
        z = (p_ref[0] + p_ref[1]) / jnp.maximum(deg, 1.0)
        o_ref[...] = jnp.maximum(z, 0.0)

    return pl.pallas_call(
        body, out_shape=jax.ShapeDtypeStruct((n, hd), jnp.float32),
    )(parts, degp)


def _tc_decoder_tail(hpre, w2, b2, w3, b3):
    e, hd = hpre.shape
    grid = e // BE

    def body(h_ref, w2_r, b2_r, w3_r, b3_r, o_ref):
        h = jnp.maximum(h_ref[...], 0.0)
        h = jnp.maximum(
            jnp.dot(h, w2_r[...], preferred_element_type=jnp.float32)
            + b2_r[...], 0.0)
        o_ref[...] = jax.nn.sigmoid(
            jnp.dot(h, w3_r[...], preferred_element_type=jnp.float32)
            + b3_r[...])

    wfull = lambda a: pl.BlockSpec(a.shape, lambda i: (0,) * a.ndim)
    b2c, b3c = b2.reshape(1, -1), b3.reshape(1, -1)
    return pl.pallas_call(
        body,
        grid=(grid,),
        in_specs=[pl.BlockSpec((BE, hd), lambda i: (i, 0)),
                  wfull(w2), wfull(b2c), wfull(w3), wfull(b3c)],
        out_specs=pl.BlockSpec((BE, 1), lambda i: (i, 0)),
        out_shape=jax.ShapeDtypeStruct((e, 1), jnp.float32),
    )(hpre, w2, b2c, w3, b3c)


def kernel(x, edge_index, region, W1, b1, We11, be11, We12, be12,
           W2, b2, We21, be21, We22, be22,
           Wd1, bd1, Wd2, bd2, Wd3, bd3):
    hd = W1.shape[1]
    px = x[:, 0]
    py = x[:, 1]
    feat = x[:, 2:]
    rg = region[:, 0]
    src = edge_index[0]
    dst = edge_index[1]

    ex, ey, er, degp = _sc_edge_features(px, py, rg, src, dst)
    h1 = _tc_linear(feat, W1, b1)
    g1, g2 = _tc_gates(ex, ey, er, We11, be11, We12, be12,
                       We21, be21, We22, be22)
    p1 = _sc_msg_agg(h1, g1, src, dst)
    z1 = _tc_z(p1, degp)
    h2 = _tc_linear(z1, W2, b2)
    p2 = _sc_msg_agg(h2, g2, src, dst)
    z2 = _tc_z(p2, degp)
    u = _tc_linear(z2, Wd1[:hd], bd1)
    v = _tc_linear(z2, Wd1[hd:], jnp.zeros((hd,), jnp.float32))
    hpre = _sc_pair_gather_add(u, v, src, dst)
    out = _tc_decoder_tail(hpre, Wd2, bd2, Wd3, bd3)
    return out.reshape(-1)


# speedup vs baseline: 2.0367x; 2.0367x over previous
import functools

import jax
import jax.numpy as jnp
from jax import lax
from jax.experimental import pallas as pl
from jax.experimental.pallas import tpu as pltpu
from jax.experimental.pallas import tpu_sc as plsc

NC, NS, L = 2, 16, 16
NW = NC * NS
K = 80
RW = 1000
BE = 2000


def _sc_mesh():
    return plsc.VectorSubcoreMesh(core_axis_name="c", subcore_axis_name="s")


_SC_PARAMS = pltpu.CompilerParams(needs_layout_passes=False)


def _sc_edge_features(px, py, rg, src, dst):
    n = px.shape[0]
    e = src.shape[0]
    per_w = e // NW
    n_chunks = per_w // K

    @functools.partial(
        pl.kernel,
        out_type=(
            jax.ShapeDtypeStruct((e,), jnp.float32),
            jax.ShapeDtypeStruct((e,), jnp.float32),
            jax.ShapeDtypeStruct((e,), jnp.float32),
        ),
        mesh=_sc_mesh(),
        compiler_params=_SC_PARAMS,
        scratch_types=[
            pltpu.VMEM((n,), jnp.float32),
            pltpu.VMEM((n,), jnp.float32),
            pltpu.VMEM((n,), jnp.float32),
            pltpu.VMEM((K,), jnp.int32),
            pltpu.VMEM((K,), jnp.int32),
            pltpu.VMEM((K,), jnp.float32),
            pltpu.VMEM((K,), jnp.float32),
            pltpu.VMEM((K,), jnp.float32),
        ],
    )
    def kern(px_h, py_h, rg_h, src_h, dst_h, ex_h, ey_h, er_h,
             px_v, py_v, rg_v, idx_s, idx_d, ex_v, ey_v, er_v):
        cid = lax.axis_index("c")
        sid = lax.axis_index("s")
        wid = sid * NC + cid
        pltpu.sync_copy(px_h, px_v)
        pltpu.sync_copy(py_h, py_v)
        pltpu.sync_copy(rg_h, rg_v)
        base_w = wid * per_w

        @pl.loop(0, n_chunks)
        def _(i):
            base = pl.multiple_of(base_w + i * K, 16)
            pltpu.sync_copy(src_h.at[pl.ds(base, K)], idx_s)
            pltpu.sync_copy(dst_h.at[pl.ds(base, K)], idx_d)
            for j in range(K // L):
                s = idx_s[pl.ds(j * L, L)]
                d = idx_d[pl.ds(j * L, L)]
                pxs = plsc.load_gather(px_v, [s])
                pxd = plsc.load_gather(px_v, [d])
                pys = plsc.load_gather(py_v, [s])
                pyd = plsc.load_gather(py_v, [d])
                rgs = plsc.load_gather(rg_v, [s])
                ex_v[pl.ds(j * L, L)] = pxs - pxd
                ey_v[pl.ds(j * L, L)] = pys - pyd
                er_v[pl.ds(j * L, L)] = rgs
            pltpu.sync_copy(ex_v, ex_h.at[pl.ds(base, K)])
            pltpu.sync_copy(ey_v, ey_h.at[pl.ds(base, K)])
            pltpu.sync_copy(er_v, er_h.at[pl.ds(base, K)])

    return kern(px, py, rg, src, dst)


def _sc_degree(dst, zin, n):
    e = dst.shape[0]
    per_w = e // NW
    n_chunks = per_w // K
    ZR = 2000

    @functools.partial(
        pl.kernel,
        out_type=jax.ShapeDtypeStruct((NC, n, 16), jnp.float32),
        mesh=_sc_mesh(),
        compiler_params=_SC_PARAMS,
        scratch_types=[
            pltpu.VMEM((K,), jnp.int32),
            pltpu.VMEM((K, 16), jnp.float32),
            pltpu.VMEM_SHARED((n, 16), jnp.float32),
        ],
    )
    def kern(dst_h, zin_h, degp_h, idx_d, ones_v, deg_sh):
        cid = lax.axis_index("c")
        sid = lax.axis_index("s")
        wid = sid * NC + cid
        ones16 = jnp.ones((L,), jnp.float32)

        @pl.loop(0, K)
        def _(i):
            ones_v[i, :] = ones16

        @pl.when(sid == NS - 1)
        def _():
            pltpu.sync_copy(zin_h, deg_sh)

        plsc.subcore_barrier()
        base_w = wid * per_w

        @pl.loop(0, n_chunks)
        def _(i):
            base = pl.multiple_of(base_w + i * K, 16)
            pltpu.sync_copy(dst_h.at[pl.ds(base, K)], idx_d)
            pltpu.sync_copy(ones_v, deg_sh.at[idx_d], add=True)

        plsc.subcore_barrier()
        for c in range(n // ZR):
            @pl.when(sid == 10 + c)
            def _(c=c):
                pltpu.sync_copy(deg_sh.at[pl.ds(c * ZR, ZR), :],
                                degp_h.at[cid, pl.ds(c * ZR, ZR), :])

    return kern(dst, zin)


def _sc_msg_agg(hp, gate, src, dst, n):
    hd = hp.shape[1]
    e = src.shape[0]
    n2 = n // NC
    na = n2 + 120
    per_t = e // NS
    n_chunks = per_t // K
    hl = hd // L

    @functools.partial(
        pl.kernel,
        out_type=jax.ShapeDtypeStruct((n, hd), jnp.float32),
        mesh=_sc_mesh(),
        compiler_params=_SC_PARAMS,
        scratch_types=[
            pltpu.VMEM((K,), jnp.int32),
            pltpu.VMEM((K,), jnp.int32),
            pltpu.VMEM((K,), jnp.int32),
            pltpu.VMEM((K, hd), jnp.float32),
            pltpu.VMEM((K, hd), jnp.float32),
            pltpu.VMEM((640, hd), jnp.float32),
            pltpu.VMEM_SHARED((na, hd), jnp.float32),
            pltpu.SemaphoreType.DMA,
        ],
    )
    def kern(h_h, gate_h, src_h, dst_h, out_h,
             idx_s, idx_d, idx_m, rows_v, gate_v, zbuf, agg_sh, sem):
        cid = lax.axis_index("c")
        sid = lax.axis_index("s")
        zeros16 = jnp.zeros((L,), jnp.float32)
        lo = cid * n2

        @pl.loop(0, 640)
        def _(i):
            for j in range(hl):
                zbuf[i, pl.ds(j * L, L)] = zeros16

        for c in range(na // 640):
            @pl.when(sid == c)
            def _(c=c):
                pltpu.sync_copy(zbuf, agg_sh.at[pl.ds(c * 640, 640), :])

        plsc.subcore_barrier()
        base_t = sid * per_t

        @pl.loop(0, n_chunks)
        def _(i):
            base = pl.multiple_of(base_t + i * K, 16)
            pltpu.sync_copy(src_h.at[pl.ds(base, K)], idx_s)
            pltpu.sync_copy(dst_h.at[pl.ds(base, K)], idx_d)
            g = pltpu.async_copy(h_h.at[idx_s], rows_v, sem)
            pltpu.sync_copy(gate_h.at[pl.ds(base, K), :], gate_v)
            for j in range(K // L):
                sl = pl.ds(j * L, L)
                d = idx_d[sl] - lo
                mine = jnp.logical_and(d >= 0, d < n2)
                idx_m[sl] = jnp.where(mine, d, n2)
            g.wait()

            @pl.loop(0, K)
            def _(r):
                for j in range(hl):
                    sl = pl.ds(j * L, L)
                    rows_v[r, sl] = rows_v[r, sl] * gate_v[r, sl]

            pltpu.sync_copy(rows_v, agg_sh.at[idx_m], add=True)

        plsc.subcore_barrier()
        for c in range(n2 // RW):
            @pl.when(sid == 8 + c)
            def _(c=c):
                off = pl.multiple_of(lo + c * RW, 8)
                pltpu.sync_copy(agg_sh.at[pl.ds(c * RW, RW), :],
                                out_h.at[pl.ds(off, RW), :])

    return kern(hp, gate, src, dst)


def _sc_msg_rows(hp, gate, src):
    hd = hp.shape[1]
    e = src.shape[0]
    per_w = e // NW
    n_chunks = per_w // K
    hl = hd // L

    @functools.partial(
        pl.kernel,
        out_type=jax.ShapeDtypeStruct((e, hd), jnp.float32),
        mesh=_sc_mesh(),
        compiler_params=_SC_PARAMS,
        scratch_types=[
            pltpu.VMEM((K,), jnp.int32),
            pltpu.VMEM((K, hd), jnp.float32),
            pltpu.VMEM((K, hd), jnp.float32),
            pltpu.SemaphoreType.DMA,
        ],
    )
    def kern(h_h, gate_h, src_h, out_h, idx_s, rows_v, gate_v, sem):
        cid = lax.axis_index("c")
        sid = lax.axis_index("s")
        wid = sid * NC + cid
        base_w = wid * per_w

        @pl.loop(0, n_chunks)
        def _(i):
            base = pl.multiple_of(base_w + i * K, 16)
            pltpu.sync_copy(src_h.at[pl.ds(base, K)], idx_s)
            g = pltpu.async_copy(h_h.at[idx_s], rows_v, sem)
            pltpu.sync_copy(gate_h.at[pl.ds(base, K), :], gate_v)
            g.wait()

            @pl.loop(0, K)
            def _(r):
                for j in range(hl):
                    sl = pl.ds(j * L, L)
                    rows_v[r, sl] = rows_v[r, sl] * gate_v[r, sl]

            pltpu.sync_copy(rows_v, out_h.at[pl.ds(base, K), :])

    return kern(hp, gate, src)


def _sc_pair_gather_add(u, v, src, dst):
    hd = u.shape[1]
    e = src.shape[0]
    per_w = e // NW
    n_chunks = per_w // K
    hl = hd // L

    @functools.partial(
        pl.kernel,
        out_type=jax.ShapeDtypeStruct((e, hd), jnp.float32),
        mesh=_sc_mesh(),
        compiler_params=_SC_PARAMS,
        scratch_types=[
            pltpu.VMEM((K,), jnp.int32),
            pltpu.VMEM((K,), jnp.int32),
            pltpu.VMEM((K, hd), jnp.float32),
            pltpu.VMEM((K, hd), jnp.float32),
            pltpu.SemaphoreType.DMA,
            pltpu.SemaphoreType.DMA,
        ],
    )
    def kern(u_h, v_h, src_h, dst_h, out_h,
             idx_s, idx_d, rows_u, rows_v, sem_u, sem_v):
        cid = lax.axis_index("c")
        sid = lax.axis_index("s")
        wid = sid * NC + cid
        base_w = wid * per_w

        @pl.loop(0, n_chunks)
        def _(i):
            base = pl.multiple_of(base_w + i * K, 16)
            pltpu.sync_copy(src_h.at[pl.ds(base, K)], idx_s)
            pltpu.sync_copy(dst_h.at[pl.ds(base, K)], idx_d)
            gu = pltpu.async_copy(u_h.at[idx_s], rows_u, sem_u)
            gv = pltpu.async_copy(v_h.at[idx_d], rows_v, sem_v)
            gu.wait()
            gv.wait()

            @pl.loop(0, K)
            def _(r):
                for j in range(hl):
                    sl = pl.ds(j * L, L)
                    rows_u[r, sl] = rows_u[r, sl] + rows_v[r, sl]

            pltpu.sync_copy(rows_u, out_h.at[pl.ds(base, K), :])

    return kern(u, v, src, dst)


def _tc_linear(x, w, b, relu=False):
    n = x.shape[0]
    m = w.shape[1]

    def body(x_ref, w_ref, b_ref, o_ref):
        y = jnp.dot(x_ref[...], w_ref[...],
                    preferred_element_type=jnp.float32) + b_ref[...]
        if relu:
            y = jnp.maximum(y, 0.0)
        o_ref[...] = y

    return pl.pallas_call(
        body, out_shape=jax.ShapeDtypeStruct((n, m), jnp.float32),
    )(x, w, b.reshape(1, -1))


def _tc_gates(ex, ey, er, w11, b11, w12, b12, w21, b21, w22, b22):
    e = ex.shape[0]
    hd = w12.shape[1]
    grid = e // BE
    exc = ex.reshape(e, 1)
    eyc = ey.reshape(e, 1)
    erc = er.reshape(e, 1)

    def body(ex_ref, ey_ref, er_ref, w11_r, b11_r, w12_r, b12_r,
             w21_r, b21_r, w22_r, b22_r, g1_ref, g2_ref):
        exb = ex_ref[...]
        eyb = ey_ref[...]
        erb = er_ref[...]
        pre1 = (exb * w11_r[0:1, :] + eyb * w11_r[1:2, :]
                + erb * w11_r[2:3, :] + b11_r[...])
        g1_ref[...] = jax.nn.sigmoid(
            jnp.dot(jnp.maximum(pre1, 0.0), w12_r[...],
                    preferred_element_type=jnp.float32) + b12_r[...])
        pre2 = (exb * w21_r[0:1, :] + eyb * w21_r[1:2, :]
                + erb * w21_r[2:3, :] + b21_r[...])
        g2_ref[...] = jax.nn.sigmoid(
            jnp.dot(jnp.maximum(pre2, 0.0), w22_r[...],
                    preferred_element_type=jnp.float32) + b22_r[...])

    col = pl.BlockSpec((BE, 1), lambda i: (i, 0))
    wfull = lambda a: pl.BlockSpec(a.shape, lambda i: (0,) * a.ndim)
    outspec = pl.BlockSpec((BE, hd), lambda i: (i, 0))
    b11c, b12c = b11.reshape(1, -1), b12.reshape(1, -1)
    b21c, b22c = b21.reshape(1, -1), b22.reshape(1, -1)
    return pl.pallas_call(
        body,
        grid=(grid,),
        in_specs=[col, col, col,
                  wfull(w11), wfull(b11c), wfull(w12), wfull(b12c),
                  wfull(w21), wfull(b21c), wfull(w22), wfull(b22c)],
        out_specs=[outspec, outspec],
        out_shape=[jax.ShapeDtypeStruct((e, hd), jnp.float32),
                   jax.ShapeDtypeStruct((e, hd), jnp.float32)],
    )(exc, eyc, erc, w11, b11c, w12, b12c, w21, b21c, w22, b22c)


def _tc_z(agg, degp):
    n, hd = agg.shape

    def body(p_ref, d_ref, o_ref):
        deg = d_ref[0, :, 0:1] + d_ref[1, :, 0:1]
        z = p_ref[...] / jnp.maximum(deg, 1.0)
        o_ref[...] = jnp.maximum(z, 0.0)

    return pl.pallas_call(
        body, out_shape=jax.ShapeDtypeStruct((n, hd), jnp.float32),
    )(agg, degp)


def _tc_zdiv(agg, degc):
    n, hd = agg.shape

    def body(p_ref, d_ref, o_ref):
        o_ref[...] = jnp.maximum(p_ref[...] / d_ref[...], 0.0)

    return pl.pallas_call(
        body, out_shape=jax.ShapeDtypeStruct((n, hd), jnp.float32),
    )(agg, degc)


def _tc_decoder_tail(hpre, w2, b2, w3, b3):
    e, hd = hpre.shape
    grid = e // BE

    def body(h_ref, w2_r, b2_r, w3_r, b3_r, o_ref):
        h = jnp.maximum(h_ref[...], 0.0)
        h = jnp.maximum(
            jnp.dot(h, w2_r[...], preferred_element_type=jnp.float32)
            + b2_r[...], 0.0)
        o_ref[...] = jax.nn.sigmoid(
            jnp.dot(h, w3_r[...], preferred_element_type=jnp.float32)
            + b3_r[...])

    wfull = lambda a: pl.BlockSpec(a.shape, lambda i: (0,) * a.ndim)
    b2c, b3c = b2.reshape(1, -1), b3.reshape(1, -1)
    return pl.pallas_call(
        body,
        grid=(grid,),
        in_specs=[pl.BlockSpec((BE, hd), lambda i: (i, 0)),
                  wfull(w2), wfull(b2c), wfull(w3), wfull(b3c)],
        out_specs=pl.BlockSpec((BE, 1), lambda i: (i, 0)),
        out_shape=jax.ShapeDtypeStruct((e, 1), jnp.float32),
    )(hpre, w2, b2c, w3, b3c)


def _pad_rows(h):
    return jnp.concatenate([h, jnp.zeros_like(h)], axis=0)


def kernel(x, edge_index, region, W1, b1, We11, be11, We12, be12,
           W2, b2, We21, be21, We22, be22,
           Wd1, bd1, Wd2, bd2, Wd3, bd3):
    n = x.shape[0]
    hd = W1.shape[1]
    px = x[:, 0]
    py = x[:, 1]
    feat = x[:, 2:]
    rg = region[:, 0]
    src = edge_index[0]
    dst = edge_index[1]

    ex, ey, er = _sc_edge_features(px, py, rg, src, dst)
    deg = jax.ops.segment_sum(jnp.ones((src.shape[0],), jnp.float32),
                              dst, num_segments=n)
    degc = jnp.maximum(deg, 1.0).reshape(n, 1)
    h1 = _tc_linear(feat, W1, b1)
    g1, g2 = _tc_gates(ex, ey, er, We11, be11, We12, be12,
                       We21, be21, We22, be22)
    m1 = _sc_msg_rows(_pad_rows(h1), g1, src)
    a1 = jax.ops.segment_sum(m1, dst, num_segments=n)
    z1 = _tc_zdiv(a1, degc)
    h2 = _tc_linear(z1, W2, b2)
    m2 = _sc_msg_rows(_pad_rows(h2), g2, src)
    a2 = jax.ops.segment_sum(m2, dst, num_segments=n)
    z2 = _tc_zdiv(a2, degc)
    u = _tc_linear(z2, Wd1[:hd], bd1)
    v = _tc_linear(z2, Wd1[hd:], jnp.zeros((hd,), jnp.float32))
    hpre = _sc_pair_gather_add(u, v, src, dst)
    out = _tc_decoder_tail(hpre, Wd2, bd2, Wd3, bd3)
    return out.reshape(-1)


# speedup vs baseline: 2.1234x; 1.0426x over previous
import functools

import jax
import jax.numpy as jnp
from jax import lax
from jax.experimental import pallas as pl
from jax.experimental.pallas import tpu as pltpu
from jax.experimental.pallas import tpu_sc as plsc

NC, NS, L = 2, 16, 16
NW = NC * NS
K = 80
RW = 1000
BE = 2000


def _sc_mesh():
    return plsc.VectorSubcoreMesh(core_axis_name="c", subcore_axis_name="s")


_SC_PARAMS = pltpu.CompilerParams(needs_layout_passes=False)


def _sc_edge_features(px, py, rg, src, dst):
    n = px.shape[0]
    e = src.shape[0]
    per_w = e // NW
    n_chunks = per_w // K

    @functools.partial(
        pl.kernel,
        out_type=(
            jax.ShapeDtypeStruct((e,), jnp.float32),
            jax.ShapeDtypeStruct((e,), jnp.float32),
            jax.ShapeDtypeStruct((e,), jnp.float32),
        ),
        mesh=_sc_mesh(),
        compiler_params=_SC_PARAMS,
        scratch_types=[
            pltpu.VMEM((n,), jnp.float32),
            pltpu.VMEM((n,), jnp.float32),
            pltpu.VMEM((n,), jnp.float32),
            pltpu.VMEM((K,), jnp.int32),
            pltpu.VMEM((K,), jnp.int32),
            pltpu.VMEM((K,), jnp.float32),
            pltpu.VMEM((K,), jnp.float32),
            pltpu.VMEM((K,), jnp.float32),
        ],
    )
    def kern(px_h, py_h, rg_h, src_h, dst_h, ex_h, ey_h, er_h,
             px_v, py_v, rg_v, idx_s, idx_d, ex_v, ey_v, er_v):
        cid = lax.axis_index("c")
        sid = lax.axis_index("s")
        wid = sid * NC + cid
        pltpu.sync_copy(px_h, px_v)
        pltpu.sync_copy(py_h, py_v)
        pltpu.sync_copy(rg_h, rg_v)
        base_w = wid * per_w

        @pl.loop(0, n_chunks)
        def _(i):
            base = pl.multiple_of(base_w + i * K, 16)
            pltpu.sync_copy(src_h.at[pl.ds(base, K)], idx_s)
            pltpu.sync_copy(dst_h.at[pl.ds(base, K)], idx_d)
            for j in range(K // L):
                s = idx_s[pl.ds(j * L, L)]
                d = idx_d[pl.ds(j * L, L)]
                pxs = plsc.load_gather(px_v, [s])
                pxd = plsc.load_gather(px_v, [d])
                pys = plsc.load_gather(py_v, [s])
                pyd = plsc.load_gather(py_v, [d])
                rgs = plsc.load_gather(rg_v, [s])
                ex_v[pl.ds(j * L, L)] = pxs - pxd
                ey_v[pl.ds(j * L, L)] = pys - pyd
                er_v[pl.ds(j * L, L)] = rgs
            pltpu.sync_copy(ex_v, ex_h.at[pl.ds(base, K)])
            pltpu.sync_copy(ey_v, ey_h.at[pl.ds(base, K)])
            pltpu.sync_copy(er_v, er_h.at[pl.ds(base, K)])

    return kern(px, py, rg, src, dst)


def _sc_degree(dst, zin, n):
    e = dst.shape[0]
    per_w = e // NW
    n_chunks = per_w // K
    ZR = 2000

    @functools.partial(
        pl.kernel,
        out_type=jax.ShapeDtypeStruct((NC, n, 16), jnp.float32),
        mesh=_sc_mesh(),
        compiler_params=_SC_PARAMS,
        scratch_types=[
            pltpu.VMEM((K,), jnp.int32),
            pltpu.VMEM((K, 16), jnp.float32),
            pltpu.VMEM_SHARED((n, 16), jnp.float32),
        ],
    )
    def kern(dst_h, zin_h, degp_h, idx_d, ones_v, deg_sh):
        cid = lax.axis_index("c")
        sid = lax.axis_index("s")
        wid = sid * NC + cid
        ones16 = jnp.ones((L,), jnp.float32)

        @pl.loop(0, K)
        def _(i):
            ones_v[i, :] = ones16

        @pl.when(sid == NS - 1)
        def _():
            pltpu.sync_copy(zin_h, deg_sh)

        plsc.subcore_barrier()
        base_w = wid * per_w

        @pl.loop(0, n_chunks)
        def _(i):
            base = pl.multiple_of(base_w + i * K, 16)
            pltpu.sync_copy(dst_h.at[pl.ds(base, K)], idx_d)
            pltpu.sync_copy(ones_v, deg_sh.at[idx_d], add=True)

        plsc.subcore_barrier()
        for c in range(n // ZR):
            @pl.when(sid == 10 + c)
            def _(c=c):
                pltpu.sync_copy(deg_sh.at[pl.ds(c * ZR, ZR), :],
                                degp_h.at[cid, pl.ds(c * ZR, ZR), :])

    return kern(dst, zin)


def _sc_msg_agg(hp, gate, src, dst, n):
    hd = hp.shape[1]
    e = src.shape[0]
    n2 = n // NC
    na = n2 + 120
    per_t = e // NS
    n_chunks = per_t // K
    hl = hd // L

    @functools.partial(
        pl.kernel,
        out_type=jax.ShapeDtypeStruct((n, hd), jnp.float32),
        mesh=_sc_mesh(),
        compiler_params=_SC_PARAMS,
        scratch_types=[
            pltpu.VMEM((K,), jnp.int32),
            pltpu.VMEM((K,), jnp.int32),
            pltpu.VMEM((K,), jnp.int32),
            pltpu.VMEM((K, hd), jnp.float32),
            pltpu.VMEM((K, hd), jnp.float32),
            pltpu.VMEM((640, hd), jnp.float32),
            pltpu.VMEM_SHARED((na, hd), jnp.float32),
            pltpu.SemaphoreType.DMA,
        ],
    )
    def kern(h_h, gate_h, src_h, dst_h, out_h,
             idx_s, idx_d, idx_m, rows_v, gate_v, zbuf, agg_sh, sem):
        cid = lax.axis_index("c")
        sid = lax.axis_index("s")
        zeros16 = jnp.zeros((L,), jnp.float32)
        lo = cid * n2

        @pl.loop(0, 640)
        def _(i):
            for j in range(hl):
                zbuf[i, pl.ds(j * L, L)] = zeros16

        for c in range(na // 640):
            @pl.when(sid == c)
            def _(c=c):
                pltpu.sync_copy(zbuf, agg_sh.at[pl.ds(c * 640, 640), :])

        plsc.subcore_barrier()
        base_t = sid * per_t

        @pl.loop(0, n_chunks)
        def _(i):
            base = pl.multiple_of(base_t + i * K, 16)
            pltpu.sync_copy(src_h.at[pl.ds(base, K)], idx_s)
            pltpu.sync_copy(dst_h.at[pl.ds(base, K)], idx_d)
            g = pltpu.async_copy(h_h.at[idx_s], rows_v, sem)
            pltpu.sync_copy(gate_h.at[pl.ds(base, K), :], gate_v)
            for j in range(K // L):
                sl = pl.ds(j * L, L)
                d = idx_d[sl] - lo
                mine = jnp.logical_and(d >= 0, d < n2)
                idx_m[sl] = jnp.where(mine, d, n2)
            g.wait()

            @pl.loop(0, K)
            def _(r):
                for j in range(hl):
                    sl = pl.ds(j * L, L)
                    rows_v[r, sl] = rows_v[r, sl] * gate_v[r, sl]

            pltpu.sync_copy(rows_v, agg_sh.at[idx_m], add=True)

        plsc.subcore_barrier()
        for c in range(n2 // RW):
            @pl.when(sid == 8 + c)
            def _(c=c):
                off = pl.multiple_of(lo + c * RW, 8)
                pltpu.sync_copy(agg_sh.at[pl.ds(c * RW, RW), :],
                                out_h.at[pl.ds(off, RW), :])

    return kern(hp, gate, src, dst)


def _sc_msg_rows(hp, gate, src):
    hd = hp.shape[1]
    e = src.shape[0]
    per_w = e // NW
    n_chunks = per_w // K
    hl = hd // L

    @functools.partial(
        pl.kernel,
        out_type=jax.ShapeDtypeStruct((e, hd), jnp.float32),
        mesh=_sc_mesh(),
        compiler_params=_SC_PARAMS,
        scratch_types=[
            pltpu.VMEM((K,), jnp.int32),
            pltpu.VMEM((K,), jnp.int32),
            pltpu.VMEM((K, hd), jnp.float32),
            pltpu.VMEM((K, hd), jnp.float32),
            pltpu.VMEM((K, hd), jnp.float32),
            pltpu.VMEM((K, hd), jnp.float32),
            pltpu.SemaphoreType.DMA,
            pltpu.SemaphoreType.DMA,
            pltpu.SemaphoreType.DMA,
            pltpu.SemaphoreType.DMA,
        ],
    )
    def kern(h_h, gate_h, src_h, out_h,
             idx0, idx1, rows0, rows1, gate0, gate1, sg0, sg1, st0, st1):
        cid = lax.axis_index("c")
        sid = lax.axis_index("s")
        wid = sid * NC + cid
        base_w = wid * per_w

        def chunk_base(i):
            return pl.multiple_of(jnp.minimum(base_w + i * K, e - K), 16)

        def issue(i, idxb, rowsb, gateb, sg, st):
            base = chunk_base(i)
            pltpu.sync_copy(src_h.at[pl.ds(base, K)], idxb)
            pltpu.async_copy(h_h.at[idxb], rowsb, sg)
            pltpu.async_copy(gate_h.at[pl.ds(base, K), :], gateb, st)

        def finish(i, idxb, rowsb, gateb, sg, st):
            base = chunk_base(i)
            pltpu.make_async_copy(h_h.at[idxb], rowsb, sg).wait()
            pltpu.make_async_copy(gate_h.at[pl.ds(base, K), :], gateb, st).wait()

            @pl.loop(0, K)
            def _(r):
                for j in range(hl):
                    sl = pl.ds(j * L, L)
                    rowsb[r, sl] = rowsb[r, sl] * gateb[r, sl]

            pltpu.sync_copy(rowsb, out_h.at[pl.ds(base, K), :])

        issue(0, idx0, rows0, gate0, sg0, st0)

        @pl.loop(0, n_chunks // 2)
        def _(ii):
            i0 = 2 * ii
            issue(i0 + 1, idx1, rows1, gate1, sg1, st1)
            finish(i0, idx0, rows0, gate0, sg0, st0)
            issue(i0 + 2, idx0, rows0, gate0, sg0, st0)
            finish(i0 + 1, idx1, rows1, gate1, sg1, st1)

        finish(n_chunks - 1, idx0, rows0, gate0, sg0, st0)

    return kern(hp, gate, src)


def _sc_pair_gather_add(u, v, src, dst):
    hd = u.shape[1]
    e = src.shape[0]
    per_w = e // NW
    n_chunks = per_w // K
    hl = hd // L

    @functools.partial(
        pl.kernel,
        out_type=jax.ShapeDtypeStruct((e, hd), jnp.float32),
        mesh=_sc_mesh(),
        compiler_params=_SC_PARAMS,
        scratch_types=[
            pltpu.VMEM((K,), jnp.int32),
            pltpu.VMEM((K,), jnp.int32),
            pltpu.VMEM((K, hd), jnp.float32),
            pltpu.VMEM((K, hd), jnp.float32),
            pltpu.SemaphoreType.DMA,
            pltpu.SemaphoreType.DMA,
        ],
    )
    def kern(u_h, v_h, src_h, dst_h, out_h,
             idx_s, idx_d, rows_u, rows_v, sem_u, sem_v):
        cid = lax.axis_index("c")
        sid = lax.axis_index("s")
        wid = sid * NC + cid
        base_w = wid * per_w

        @pl.loop(0, n_chunks)
        def _(i):
            base = pl.multiple_of(base_w + i * K, 16)
            pltpu.sync_copy(src_h.at[pl.ds(base, K)], idx_s)
            pltpu.sync_copy(dst_h.at[pl.ds(base, K)], idx_d)
            gu = pltpu.async_copy(u_h.at[idx_s], rows_u, sem_u)
            gv = pltpu.async_copy(v_h.at[idx_d], rows_v, sem_v)
            gu.wait()
            gv.wait()

            @pl.loop(0, K)
            def _(r):
                for j in range(hl):
                    sl = pl.ds(j * L, L)
                    rows_u[r, sl] = rows_u[r, sl] + rows_v[r, sl]

            pltpu.sync_copy(rows_u, out_h.at[pl.ds(base, K), :])

    return kern(u, v, src, dst)


def _tc_linear(x, w, b, relu=False):
    n = x.shape[0]
    m = w.shape[1]

    def body(x_ref, w_ref, b_ref, o_ref):
        y = jnp.dot(x_ref[...], w_ref[...],
                    preferred_element_type=jnp.float32) + b_ref[...]
        if relu:
            y = jnp.maximum(y, 0.0)
        o_ref[...] = y

    return pl.pallas_call(
        body, out_shape=jax.ShapeDtypeStruct((n, m), jnp.float32),
    )(x, w, b.reshape(1, -1))


def _tc_gates(ex, ey, er, w11, b11, w12, b12, w21, b21, w22, b22):
    e = ex.shape[0]
    hd = w12.shape[1]
    grid = e // BE
    exc = ex.reshape(e, 1)
    eyc = ey.reshape(e, 1)
    erc = er.reshape(e, 1)

    def body(ex_ref, ey_ref, er_ref, w11_r, b11_r, w12_r, b12_r,
             w21_r, b21_r, w22_r, b22_r, g1_ref, g2_ref):
        exb = ex_ref[...]
        eyb = ey_ref[...]
        erb = er_ref[...]
        pre1 = (exb * w11_r[0:1, :] + eyb * w11_r[1:2, :]
                + erb * w11_r[2:3, :] + b11_r[...])
        g1_ref[...] = jax.nn.sigmoid(
            jnp.dot(jnp.maximum(pre1, 0.0), w12_r[...],
                    preferred_element_type=jnp.float32) + b12_r[...])
        pre2 = (exb * w21_r[0:1, :] + eyb * w21_r[1:2, :]
                + erb * w21_r[2:3, :] + b21_r[...])
        g2_ref[...] = jax.nn.sigmoid(
            jnp.dot(jnp.maximum(pre2, 0.0), w22_r[...],
                    preferred_element_type=jnp.float32) + b22_r[...])

    col = pl.BlockSpec((BE, 1), lambda i: (i, 0))
    wfull = lambda a: pl.BlockSpec(a.shape, lambda i: (0,) * a.ndim)
    outspec = pl.BlockSpec((BE, hd), lambda i: (i, 0))
    b11c, b12c = b11.reshape(1, -1), b12.reshape(1, -1)
    b21c, b22c = b21.reshape(1, -1), b22.reshape(1, -1)
    return pl.pallas_call(
        body,
        grid=(grid,),
        in_specs=[col, col, col,
                  wfull(w11), wfull(b11c), wfull(w12), wfull(b12c),
                  wfull(w21), wfull(b21c), wfull(w22), wfull(b22c)],
        out_specs=[outspec, outspec],
        out_shape=[jax.ShapeDtypeStruct((e, hd), jnp.float32),
                   jax.ShapeDtypeStruct((e, hd), jnp.float32)],
    )(exc, eyc, erc, w11, b11c, w12, b12c, w21, b21c, w22, b22c)


def _tc_z(agg, degp):
    n, hd = agg.shape

    def body(p_ref, d_ref, o_ref):
        deg = d_ref[0, :, 0:1] + d_ref[1, :, 0:1]
        z = p_ref[...] / jnp.maximum(deg, 1.0)
        o_ref[...] = jnp.maximum(z, 0.0)

    return pl.pallas_call(
        body, out_shape=jax.ShapeDtypeStruct((n, hd), jnp.float32),
    )(agg, degp)


def _tc_zdiv(agg, degc):
    n, hd = agg.shape

    def body(p_ref, d_ref, o_ref):
        o_ref[...] = jnp.maximum(p_ref[...] / d_ref[...], 0.0)

    return pl.pallas_call(
        body, out_shape=jax.ShapeDtypeStruct((n, hd), jnp.float32),
    )(agg, degc)


def _tc_decoder_tail(hpre, w2, b2, w3, b3):
    e, hd = hpre.shape
    grid = e // BE

    def body(h_ref, w2_r, b2_r, w3_r, b3_r, o_ref):
        h = jnp.maximum(h_ref[...], 0.0)
        h = jnp.maximum(
            jnp.dot(h, w2_r[...], preferred_element_type=jnp.float32)
            + b2_r[...], 0.0)
        o_ref[...] = jax.nn.sigmoid(
            jnp.dot(h, w3_r[...], preferred_element_type=jnp.float32)
            + b3_r[...])

    wfull = lambda a: pl.BlockSpec(a.shape, lambda i: (0,) * a.ndim)
    b2c, b3c = b2.reshape(1, -1), b3.reshape(1, -1)
    return pl.pallas_call(
        body,
        grid=(grid,),
        in_specs=[pl.BlockSpec((BE, hd), lambda i: (i, 0)),
                  wfull(w2), wfull(b2c), wfull(w3), wfull(b3c)],
        out_specs=pl.BlockSpec((BE, 1), lambda i: (i, 0)),
        out_shape=jax.ShapeDtypeStruct((e, 1), jnp.float32),
    )(hpre, w2, b2c, w3, b3c)


def _pad_rows(h):
    return jnp.concatenate([h, jnp.zeros_like(h)], axis=0)


def kernel(x, edge_index, region, W1, b1, We11, be11, We12, be12,
           W2, b2, We21, be21, We22, be22,
           Wd1, bd1, Wd2, bd2, Wd3, bd3):
    n = x.shape[0]
    hd = W1.shape[1]
    px = x[:, 0]
    py = x[:, 1]
    feat = x[:, 2:]
    rg = region[:, 0]
    src = edge_index[0]
    dst = edge_index[1]

    ex, ey, er = _sc_edge_features(px, py, rg, src, dst)
    deg = jax.ops.segment_sum(jnp.ones((src.shape[0],), jnp.float32),
                              dst, num_segments=n)
    degc = jnp.maximum(deg, 1.0).reshape(n, 1)
    h1 = _tc_linear(feat, W1, b1)
    g1, g2 = _tc_gates(ex, ey, er, We11, be11, We12, be12,
                       We21, be21, We22, be22)
    m1 = _sc_msg_rows(_pad_rows(h1), g1, src)
    a1 = jax.ops.segment_sum(m1, dst, num_segments=n)
    z1 = _tc_zdiv(a1, degc)
    h2 = _tc_linear(z1, W2, b2)
    m2 = _sc_msg_rows(_pad_rows(h2), g2, src)
    a2 = jax.ops.segment_sum(m2, dst, num_segments=n)
    z2 = _tc_zdiv(a2, degc)
    u = _tc_linear(z2, Wd1[:hd], bd1)
    v = _tc_linear(z2, Wd1[hd:], jnp.zeros((hd,), jnp.float32))
    hpre = _sc_pair_gather_add(u, v, src, dst)
    out = _tc_decoder_tail(hpre, Wd2, bd2, Wd3, bd3)
    return out.reshape(-1)
